# baseline (speedup 1.0000x reference)
import jax
import jax.numpy as jnp
from jax.experimental import pallas as pl
from jax.experimental.pallas import tpu as pltpu

ROWS = 128
COLS = 32768
LANE = 128
CHUNK_R = 8
NCH = ROWS // CHUNK_R
CV = COLS // LANE
DEPTH = 4
ODEPTH = 3
_BIG = 2**31 - 1


def _compute(src, dst):
    acc = src[:, 0:LANE]
    aidx = jnp.zeros((CHUNK_R, LANE), jnp.int32)
    for c in range(1, CV):
        xv = src[:, c * LANE : (c + 1) * LANE]
        m = xv > acc
        acc = jnp.where(m, xv, acc)
        aidx = jnp.where(m, jnp.full((CHUNK_R, LANE), c, jnp.int32), aidx)
    rowmax = jnp.max(acc, axis=1, keepdims=True)
    lanes = jax.lax.broadcasted_iota(jnp.int32, (CHUNK_R, LANE), 1)
    gidx = aidx * LANE + lanes
    idx = jnp.min(
        jnp.where(acc == rowmax, gidx, jnp.int32(_BIG)), axis=1, keepdims=True
    )
    cols = jax.lax.broadcasted_iota(jnp.int32, (CHUNK_R, COLS), 1)
    dst[...] = jnp.where(cols == idx, 1.0, 0.0).astype(jnp.float32)


def _body(x_hbm, out_hbm, inb, outb, isems, osems):
    in_d = {}
    for i in range(min(DEPTH, NCH)):
        in_d[i] = pltpu.async_copy(
            x_hbm.at[pl.ds(i * CHUNK_R, CHUNK_R), :], inb.at[i], isems.at[i]
        )
    out_d = {}
    for i in range(NCH):
        in_d[i].wait()
        islot = i % DEPTH
        oslot = i % ODEPTH
        if i >= ODEPTH:
            out_d[i - ODEPTH].wait()
        _compute(inb.at[islot], outb.at[oslot])
        if i + DEPTH < NCH:
            in_d[i + DEPTH] = pltpu.async_copy(
                x_hbm.at[pl.ds((i + DEPTH) * CHUNK_R, CHUNK_R), :],
                inb.at[(i + DEPTH) % DEPTH],
                isems.at[(i + DEPTH) % DEPTH],
            )
        out_d[i] = pltpu.async_copy(
            outb.at[oslot], out_hbm.at[pl.ds(i * CHUNK_R, CHUNK_R), :], osems.at[oslot]
        )
    for i in range(NCH - ODEPTH, NCH):
        out_d[i].wait()


_call = pl.pallas_call(
    _body,
    in_specs=[pl.BlockSpec(memory_space=pl.ANY)],
    out_specs=pl.BlockSpec(memory_space=pl.ANY),
    out_shape=jax.ShapeDtypeStruct((ROWS, COLS), jnp.float32),
    scratch_shapes=[
        pltpu.VMEM((DEPTH, CHUNK_R, COLS), jnp.float32),
        pltpu.VMEM((ODEPTH, CHUNK_R, COLS), jnp.float32),
        pltpu.SemaphoreType.DMA((DEPTH,)),
        pltpu.SemaphoreType.DMA((ODEPTH,)),
    ],
)


def kernel(x):
    return _call(x)


# speedup vs baseline: 1.0718x; 1.0718x over previous
import jax
import jax.numpy as jnp
from jax.experimental import pallas as pl
from jax.experimental.pallas import tpu as pltpu

ROWS = 128
COLS = 32768
LANE = 128
CHUNK_R = 32
NCH = ROWS // CHUNK_R
CV = COLS // LANE
DEPTH = 2
ODEPTH = 2
_BIG = 2**31 - 1


def _compute(src, dst):
    acc = src[:, 0:LANE]
    aidx = jnp.zeros((CHUNK_R, LANE), jnp.int32)
    for c in range(1, CV):
        xv = src[:, c * LANE : (c + 1) * LANE]
        m = xv > acc
        acc = jnp.where(m, xv, acc)
        aidx = jnp.where(m, jnp.full((CHUNK_R, LANE), c, jnp.int32), aidx)
    rowmax = jnp.max(acc, axis=1, keepdims=True)
    lanes = jax.lax.broadcasted_iota(jnp.int32, (CHUNK_R, LANE), 1)
    gidx = aidx * LANE + lanes
    idx = jnp.min(
        jnp.where(acc == rowmax, gidx, jnp.int32(_BIG)), axis=1, keepdims=True
    )
    cols = jax.lax.broadcasted_iota(jnp.int32, (CHUNK_R, COLS), 1)
    dst[...] = jnp.where(cols == idx, 1.0, 0.0).astype(jnp.float32)


def _body(x_hbm, out_hbm, inb, outb, isems, osems):
    in_d = {}
    for i in range(min(DEPTH, NCH)):
        in_d[i] = pltpu.async_copy(
            x_hbm.at[pl.ds(i * CHUNK_R, CHUNK_R), :], inb.at[i], isems.at[i]
        )
    out_d = {}
    for i in range(NCH):
        in_d[i].wait()
        islot = i % DEPTH
        oslot = i % ODEPTH
        if i >= ODEPTH:
            out_d[i - ODEPTH].wait()
        _compute(inb.at[islot], outb.at[oslot])
        if i + DEPTH < NCH:
            in_d[i + DEPTH] = pltpu.async_copy(
                x_hbm.at[pl.ds((i + DEPTH) * CHUNK_R, CHUNK_R), :],
                inb.at[(i + DEPTH) % DEPTH],
                isems.at[(i + DEPTH) % DEPTH],
            )
        out_d[i] = pltpu.async_copy(
            outb.at[oslot], out_hbm.at[pl.ds(i * CHUNK_R, CHUNK_R), :], osems.at[oslot]
        )
    for i in range(NCH - ODEPTH, NCH):
        out_d[i].wait()


_call = pl.pallas_call(
    _body,
    in_specs=[pl.BlockSpec(memory_space=pl.ANY)],
    out_specs=pl.BlockSpec(memory_space=pl.ANY),
    out_shape=jax.ShapeDtypeStruct((ROWS, COLS), jnp.float32),
    scratch_shapes=[
        pltpu.VMEM((DEPTH, CHUNK_R, COLS), jnp.float32),
        pltpu.VMEM((ODEPTH, CHUNK_R, COLS), jnp.float32),
        pltpu.SemaphoreType.DMA((DEPTH,)),
        pltpu.SemaphoreType.DMA((ODEPTH,)),
    ],
)


def kernel(x):
    return _call(x)
